# baseline (speedup 1.0000x reference)
import functools

import jax
import jax.numpy as jnp
from jax import lax
from jax.experimental import pallas as pl
from jax.experimental.pallas import tpu as pltpu
from jax.experimental.pallas import tpu_sc as plsc

BATCH = 16384
VOCAB = 100000
HIER_VOCAB = 10000
EMBED_DIM = 32
OUT_DIM = 32
PAD_DIM = 128

_NC = 2
_NS = 16
_NW = _NC * _NS
_B_PER_W = BATCH // _NW
_CHUNK = 128
_NCHUNK = _B_PER_W // _CHUNK


def _sc_gather_body(idx_hbm, par_hbm, code_hbm, hier_hbm, ec_out, ep_out,
                    idx_v, par_v, c0, c1, h0, h1,
                    sc0, sc1, sh0, sh1):
    wid = lax.axis_index("s") * _NC + lax.axis_index("c")
    base = wid * _B_PER_W
    for j in range(_NCHUNK):
        pltpu.sync_copy(idx_hbm.at[pl.ds(base + j * _CHUNK, _CHUNK)], idx_v.at[j])
        pltpu.sync_copy(par_hbm.at[pl.ds(base + j * _CHUNK, _CHUNK)], par_v.at[j])
    cbufs = (c0, c1)
    hbufs = (h0, h1)
    csems = (sc0, sc1)
    hsems = (sh0, sh1)
    cps_c = [None] * _NCHUNK
    cps_h = [None] * _NCHUNK
    for j in range(2):
        cps_c[j] = pltpu.async_copy(code_hbm.at[idx_v.at[j]], cbufs[j], csems[j])
        cps_h[j] = pltpu.async_copy(hier_hbm.at[par_v.at[j]], hbufs[j], hsems[j])
    for j in range(_NCHUNK):
        rows = pl.ds(base + j * _CHUNK, _CHUNK)
        lanes = pl.ds(0, EMBED_DIM)
        cps_c[j].wait()
        pltpu.sync_copy(cbufs[j % 2], ec_out.at[rows, lanes])
        if j + 2 < _NCHUNK:
            cps_c[j + 2] = pltpu.async_copy(
                code_hbm.at[idx_v.at[j + 2]], cbufs[j % 2], csems[j % 2])
        cps_h[j].wait()
        pltpu.sync_copy(hbufs[j % 2], ep_out.at[rows, lanes])
        if j + 2 < _NCHUNK:
            cps_h[j + 2] = pltpu.async_copy(
                hier_hbm.at[par_v.at[j + 2]], hbufs[j % 2], hsems[j % 2])


_sc_gather = functools.partial(
    pl.kernel,
    out_type=[
        jax.ShapeDtypeStruct((BATCH, PAD_DIM), jnp.float32),
        jax.ShapeDtypeStruct((BATCH, PAD_DIM), jnp.float32),
    ],
    mesh=plsc.VectorSubcoreMesh(core_axis_name="c", subcore_axis_name="s"),
    compiler_params=pltpu.CompilerParams(use_tc_tiling_on_sc=False),
    scratch_types=[
        pltpu.VMEM((_NCHUNK, _CHUNK), jnp.int32),
        pltpu.VMEM((_NCHUNK, _CHUNK), jnp.int32),
        pltpu.VMEM((_CHUNK, EMBED_DIM), jnp.float32),
        pltpu.VMEM((_CHUNK, EMBED_DIM), jnp.float32),
        pltpu.VMEM((_CHUNK, EMBED_DIM), jnp.float32),
        pltpu.VMEM((_CHUNK, EMBED_DIM), jnp.float32),
        pltpu.SemaphoreType.DMA,
        pltpu.SemaphoreType.DMA,
        pltpu.SemaphoreType.DMA,
        pltpu.SemaphoreType.DMA,
    ],
)(_sc_gather_body)


_TP_BLK = 4096


def _tp_body(tt_ref, o_ref):
    t = jax.lax.dot_general(
        tt_ref[...], jnp.eye(EMBED_DIM, dtype=jnp.float32),
        (((0,), (0,)), ((), ())), preferred_element_type=jnp.float32)
    t3 = t.reshape(_TP_BLK // 4, 4, EMBED_DIM)
    for a in range(4):
        o_ref[:, pl.ds(a * EMBED_DIM, EMBED_DIM)] = t3[:, a, :]


def _tc_transpose_pack(table_t, vocab):
    grid = ((vocab + _TP_BLK - 1) // _TP_BLK,)
    return pl.pallas_call(
        _tp_body,
        grid=grid,
        in_specs=[pl.BlockSpec((EMBED_DIM, _TP_BLK), lambda i: (0, i))],
        out_specs=pl.BlockSpec((_TP_BLK // 4, PAD_DIM), lambda i: (i, 0)),
        out_shape=jax.ShapeDtypeStruct((vocab // 4, PAD_DIM), jnp.float32),
    )(table_t)


_MM_BLK = 2048


def _mm_body(ec_ref, ep_ref, w1_ref, w2_ref, b_ref, o_ref):
    ec = ec_ref[:, : EMBED_DIM]
    ep = ep_ref[:, : EMBED_DIM]
    acc = jax.lax.dot_general(
        w1_ref[...], ec, (((0,), (1,)), ((), ())),
        preferred_element_type=jnp.float32)
    acc += jax.lax.dot_general(
        w2_ref[...], ep, (((0,), (1,)), ((), ())),
        preferred_element_type=jnp.float32)
    o_ref[...] = acc + b_ref[...]


def _tc_project(ec, ep, w1, w2, bcol):
    grid = (BATCH // _MM_BLK,)
    return pl.pallas_call(
        _mm_body,
        grid=grid,
        in_specs=[
            pl.BlockSpec((_MM_BLK, PAD_DIM), lambda i: (i, 0)),
            pl.BlockSpec((_MM_BLK, PAD_DIM), lambda i: (i, 0)),
            pl.BlockSpec((EMBED_DIM, OUT_DIM), lambda i: (0, 0)),
            pl.BlockSpec((EMBED_DIM, OUT_DIM), lambda i: (0, 0)),
            pl.BlockSpec((OUT_DIM, 1), lambda i: (0, 0)),
        ],
        out_specs=pl.BlockSpec((OUT_DIM, _MM_BLK), lambda i: (0, i)),
        out_shape=jax.ShapeDtypeStruct((OUT_DIM, BATCH), jnp.float32),
    )(ec, ep, w1, w2, bcol)


@jax.jit
def kernel(indices, parents, code_table, hier_table, W, b):
    code_lin = _tc_transpose_pack(code_table.T, VOCAB).reshape(VOCAB, EMBED_DIM)
    hier_lin = _tc_transpose_pack(hier_table.T, HIER_VOCAB).reshape(
        HIER_VOCAB, EMBED_DIM)
    ec, ep = _sc_gather(indices, parents, code_lin, hier_lin)
    w1 = W[:EMBED_DIM]
    w2 = W[EMBED_DIM:]
    out_t = _tc_project(ec, ep, w1, w2, b.reshape(OUT_DIM, 1))
    return out_t.T


# speedup vs baseline: 1.1797x; 1.1797x over previous
import functools

import jax
import jax.numpy as jnp
from jax import lax
from jax.experimental import pallas as pl
from jax.experimental.pallas import tpu as pltpu
from jax.experimental.pallas import tpu_sc as plsc

BATCH = 16384
VOCAB = 100000
HIER_VOCAB = 10000
EMBED_DIM = 32
OUT_DIM = 32
PAD_DIM = 128

_NC = 2
_NS = 16
_NW = _NC * _NS
_B_PER_W = BATCH // _NW
_CHUNK = 128
_NCHUNK = _B_PER_W // _CHUNK


def _sc_gather_body(idx_hbm, par_hbm, code_hbm, hier_hbm, ec_out, ep_out,
                    idx_v, par_v, c0, c1, h0, h1,
                    sc0, sc1, sh0, sh1):
    wid = lax.axis_index("s") * _NC + lax.axis_index("c")
    base = wid * _B_PER_W
    for j in range(_NCHUNK):
        pltpu.sync_copy(idx_hbm.at[pl.ds(base + j * _CHUNK, _CHUNK)], idx_v.at[j])
        pltpu.sync_copy(par_hbm.at[pl.ds(base + j * _CHUNK, _CHUNK)], par_v.at[j])
    for ref in (idx_v, par_v):
        for j in range(_NCHUNK):
            for k in range(_CHUNK // 16):
                s = pl.ds(k * 16, 16)
                ref[j, s] = ref[j, s] * 4
    cbufs = (c0, c1)
    hbufs = (h0, h1)
    csems = (sc0, sc1)
    hsems = (sh0, sh1)
    cps_c = [None] * _NCHUNK
    cps_h = [None] * _NCHUNK
    for j in range(2):
        cps_c[j] = pltpu.async_copy(code_hbm.at[idx_v.at[j]], cbufs[j], csems[j])
        cps_h[j] = pltpu.async_copy(hier_hbm.at[par_v.at[j]], hbufs[j], hsems[j])
    for j in range(_NCHUNK):
        rows = pl.ds(base + j * _CHUNK, _CHUNK)
        lanes = pl.ds(0, EMBED_DIM)
        cps_c[j].wait()
        pltpu.sync_copy(cbufs[j % 2], ec_out.at[rows, lanes])
        if j + 2 < _NCHUNK:
            cps_c[j + 2] = pltpu.async_copy(
                code_hbm.at[idx_v.at[j + 2]], cbufs[j % 2], csems[j % 2])
        cps_h[j].wait()
        pltpu.sync_copy(hbufs[j % 2], ep_out.at[rows, lanes])
        if j + 2 < _NCHUNK:
            cps_h[j + 2] = pltpu.async_copy(
                hier_hbm.at[par_v.at[j + 2]], hbufs[j % 2], hsems[j % 2])


_sc_gather = functools.partial(
    pl.kernel,
    out_type=[
        jax.ShapeDtypeStruct((BATCH, PAD_DIM), jnp.float32),
        jax.ShapeDtypeStruct((BATCH, PAD_DIM), jnp.float32),
    ],
    mesh=plsc.VectorSubcoreMesh(core_axis_name="c", subcore_axis_name="s"),
    compiler_params=pltpu.CompilerParams(use_tc_tiling_on_sc=False),
    scratch_types=[
        pltpu.VMEM((_NCHUNK, _CHUNK), jnp.int32),
        pltpu.VMEM((_NCHUNK, _CHUNK), jnp.int32),
        pltpu.VMEM((_CHUNK, EMBED_DIM), jnp.float32),
        pltpu.VMEM((_CHUNK, EMBED_DIM), jnp.float32),
        pltpu.VMEM((_CHUNK, EMBED_DIM), jnp.float32),
        pltpu.VMEM((_CHUNK, EMBED_DIM), jnp.float32),
        pltpu.SemaphoreType.DMA,
        pltpu.SemaphoreType.DMA,
        pltpu.SemaphoreType.DMA,
        pltpu.SemaphoreType.DMA,
    ],
)(_sc_gather_body)


_TP_BLK = 4096


def _tp_body(tt_ref, o_ref):
    t = jax.lax.dot_general(
        tt_ref[...], jnp.eye(EMBED_DIM, dtype=jnp.float32),
        (((0,), (0,)), ((), ())), preferred_element_type=jnp.float32)
    o_ref[...] = jnp.concatenate(
        [t, jnp.zeros((_TP_BLK, PAD_DIM - EMBED_DIM), jnp.float32)], axis=1)


def _tc_transpose_pad(table_t, vocab):
    grid = ((vocab + _TP_BLK - 1) // _TP_BLK,)
    return pl.pallas_call(
        _tp_body,
        grid=grid,
        in_specs=[pl.BlockSpec((EMBED_DIM, _TP_BLK), lambda i: (0, i))],
        out_specs=pl.BlockSpec((_TP_BLK, PAD_DIM), lambda i: (i, 0)),
        out_shape=jax.ShapeDtypeStruct((vocab, PAD_DIM), jnp.float32),
    )(table_t)


_MM_BLK = 2048


def _mm_body(ec_ref, ep_ref, w1_ref, w2_ref, b_ref, o_ref):
    ec = ec_ref[:, : EMBED_DIM]
    ep = ep_ref[:, : EMBED_DIM]
    acc = jax.lax.dot_general(
        w1_ref[...], ec, (((0,), (1,)), ((), ())),
        preferred_element_type=jnp.float32)
    acc += jax.lax.dot_general(
        w2_ref[...], ep, (((0,), (1,)), ((), ())),
        preferred_element_type=jnp.float32)
    o_ref[...] = acc + b_ref[...]


def _tc_project(ec, ep, w1, w2, bcol):
    grid = (BATCH // _MM_BLK,)
    return pl.pallas_call(
        _mm_body,
        grid=grid,
        in_specs=[
            pl.BlockSpec((_MM_BLK, PAD_DIM), lambda i: (i, 0)),
            pl.BlockSpec((_MM_BLK, PAD_DIM), lambda i: (i, 0)),
            pl.BlockSpec((EMBED_DIM, OUT_DIM), lambda i: (0, 0)),
            pl.BlockSpec((EMBED_DIM, OUT_DIM), lambda i: (0, 0)),
            pl.BlockSpec((OUT_DIM, 1), lambda i: (0, 0)),
        ],
        out_specs=pl.BlockSpec((OUT_DIM, _MM_BLK), lambda i: (0, i)),
        out_shape=jax.ShapeDtypeStruct((OUT_DIM, BATCH), jnp.float32),
    )(ec, ep, w1, w2, bcol)


@jax.jit
def kernel(indices, parents, code_table, hier_table, W, b):
    code_lin = _tc_transpose_pad(code_table.T, VOCAB).reshape(
        4 * VOCAB, EMBED_DIM)
    hier_lin = _tc_transpose_pad(hier_table.T, HIER_VOCAB).reshape(
        4 * HIER_VOCAB, EMBED_DIM)
    ec, ep = _sc_gather(indices, parents, code_lin, hier_lin)
    w1 = W[:EMBED_DIM]
    w2 = W[EMBED_DIM:]
    out_t = _tc_project(ec, ep, w1, w2, b.reshape(OUT_DIM, 1))
    return out_t.T


# speedup vs baseline: 1.3998x; 1.1866x over previous
import functools

import jax
import jax.numpy as jnp
from jax import lax
from jax.experimental import pallas as pl
from jax.experimental.pallas import tpu as pltpu
from jax.experimental.pallas import tpu_sc as plsc

BATCH = 16384
VOCAB = 100000
HIER_VOCAB = 10000
EMBED_DIM = 32
OUT_DIM = 32
PAD_DIM = 128

_NC = 2
_NS = 16
_NW = _NC * _NS
_B_PER_W = BATCH // _NW
_CHUNK = 128
_NCHUNK = _B_PER_W // _CHUNK


def _sc_gather_body(idx_hbm, par_hbm, code_hbm, hier_hbm, ec_out, ep_out,
                    idx_v, par_v, c0, c1, h0, h1,
                    sc0, sc1, sh0, sh1):
    wid = lax.axis_index("s") * _NC + lax.axis_index("c")
    base = wid * _B_PER_W
    stage = []
    for j in range(_NCHUNK):
        src = pl.ds(base + j * _CHUNK, _CHUNK)
        stage.append(pltpu.async_copy(idx_hbm.at[src], idx_v.at[j], sc0))
        stage.append(pltpu.async_copy(par_hbm.at[src], par_v.at[j], sc1))
    for cp in stage:
        cp.wait()
    for ref in (idx_v, par_v):
        for j in range(_NCHUNK):
            for k in range(_CHUNK // 16):
                s = pl.ds(k * 16, 16)
                ref[j, s] = ref[j, s] * 4
    cbufs = (c0, c1)
    hbufs = (h0, h1)
    csems = (sc0, sc1)
    hsems = (sh0, sh1)
    cps_c = [None] * _NCHUNK
    cps_h = [None] * _NCHUNK
    for j in range(2):
        cps_c[j] = pltpu.async_copy(code_hbm.at[idx_v.at[j]], cbufs[j], csems[j])
        cps_h[j] = pltpu.async_copy(hier_hbm.at[par_v.at[j]], hbufs[j], hsems[j])
    for j in range(_NCHUNK):
        rows = pl.ds(base + j * _CHUNK, _CHUNK)
        lanes = pl.ds(0, EMBED_DIM)
        cps_c[j].wait()
        pltpu.sync_copy(cbufs[j % 2], ec_out.at[rows, lanes])
        if j + 2 < _NCHUNK:
            cps_c[j + 2] = pltpu.async_copy(
                code_hbm.at[idx_v.at[j + 2]], cbufs[j % 2], csems[j % 2])
        cps_h[j].wait()
        pltpu.sync_copy(hbufs[j % 2], ep_out.at[rows, lanes])
        if j + 2 < _NCHUNK:
            cps_h[j + 2] = pltpu.async_copy(
                hier_hbm.at[par_v.at[j + 2]], hbufs[j % 2], hsems[j % 2])


_sc_gather = functools.partial(
    pl.kernel,
    out_type=[
        jax.ShapeDtypeStruct((BATCH, PAD_DIM), jnp.float32),
        jax.ShapeDtypeStruct((BATCH, PAD_DIM), jnp.float32),
    ],
    mesh=plsc.VectorSubcoreMesh(core_axis_name="c", subcore_axis_name="s"),
    compiler_params=pltpu.CompilerParams(use_tc_tiling_on_sc=False),
    scratch_types=[
        pltpu.VMEM((_NCHUNK, _CHUNK), jnp.int32),
        pltpu.VMEM((_NCHUNK, _CHUNK), jnp.int32),
        pltpu.VMEM((_CHUNK, EMBED_DIM), jnp.float32),
        pltpu.VMEM((_CHUNK, EMBED_DIM), jnp.float32),
        pltpu.VMEM((_CHUNK, EMBED_DIM), jnp.float32),
        pltpu.VMEM((_CHUNK, EMBED_DIM), jnp.float32),
        pltpu.SemaphoreType.DMA,
        pltpu.SemaphoreType.DMA,
        pltpu.SemaphoreType.DMA,
        pltpu.SemaphoreType.DMA,
    ],
)(_sc_gather_body)


_TP_BLK = 8192


def _tp_body(tt_ref, o_ref):
    t = jax.lax.dot_general(
        tt_ref[...], jnp.eye(EMBED_DIM, dtype=jnp.float32),
        (((0,), (0,)), ((), ())), preferred_element_type=jnp.float32)
    o_ref[...] = jnp.concatenate(
        [t, jnp.zeros((_TP_BLK, PAD_DIM - EMBED_DIM), jnp.float32)], axis=1)


def _tc_transpose_pad(table_t, vocab):
    grid = ((vocab + _TP_BLK - 1) // _TP_BLK,)
    return pl.pallas_call(
        _tp_body,
        grid=grid,
        in_specs=[pl.BlockSpec((EMBED_DIM, _TP_BLK), lambda i: (0, i))],
        out_specs=pl.BlockSpec((_TP_BLK, PAD_DIM), lambda i: (i, 0)),
        out_shape=jax.ShapeDtypeStruct((vocab, PAD_DIM), jnp.float32),
    )(table_t)


_MM_BLK = 4096


def _mm_body(ec_ref, ep_ref, w1_ref, w2_ref, b_ref, o_ref):
    ec = ec_ref[:, : EMBED_DIM]
    ep = ep_ref[:, : EMBED_DIM]
    acc = jax.lax.dot_general(
        w1_ref[...], ec, (((0,), (1,)), ((), ())),
        preferred_element_type=jnp.float32)
    acc += jax.lax.dot_general(
        w2_ref[...], ep, (((0,), (1,)), ((), ())),
        preferred_element_type=jnp.float32)
    o_ref[...] = acc + b_ref[...]


def _tc_project(ec, ep, w1, w2, bcol):
    grid = (BATCH // _MM_BLK,)
    return pl.pallas_call(
        _mm_body,
        grid=grid,
        in_specs=[
            pl.BlockSpec((_MM_BLK, PAD_DIM), lambda i: (i, 0)),
            pl.BlockSpec((_MM_BLK, PAD_DIM), lambda i: (i, 0)),
            pl.BlockSpec((EMBED_DIM, OUT_DIM), lambda i: (0, 0)),
            pl.BlockSpec((EMBED_DIM, OUT_DIM), lambda i: (0, 0)),
            pl.BlockSpec((OUT_DIM, 1), lambda i: (0, 0)),
        ],
        out_specs=pl.BlockSpec((OUT_DIM, _MM_BLK), lambda i: (0, i)),
        out_shape=jax.ShapeDtypeStruct((OUT_DIM, BATCH), jnp.float32),
    )(ec, ep, w1, w2, bcol)


@jax.jit
def kernel(indices, parents, code_table, hier_table, W, b):
    code_lin = _tc_transpose_pad(code_table.T, VOCAB).reshape(
        4 * VOCAB, EMBED_DIM)
    hier_lin = _tc_transpose_pad(hier_table.T, HIER_VOCAB).reshape(
        4 * HIER_VOCAB, EMBED_DIM)
    ec, ep = _sc_gather(indices, parents, code_lin, hier_lin)
    w1 = W[:EMBED_DIM]
    w2 = W[EMBED_DIM:]
    out_t = _tc_project(ec, ep, w1, w2, b.reshape(OUT_DIM, 1))
    return out_t.T


# speedup vs baseline: 1.4570x; 1.0409x over previous
import functools

import jax
import jax.numpy as jnp
from jax import lax
from jax.experimental import pallas as pl
from jax.experimental.pallas import tpu as pltpu
from jax.experimental.pallas import tpu_sc as plsc

BATCH = 16384
VOCAB = 100000
HIER_VOCAB = 10000
EMBED_DIM = 32
OUT_DIM = 32
PAD_DIM = 128

_NC = 2
_NS = 16
_NW = _NC * _NS
_B_PER_W = BATCH // _NW
_CHUNK = 128
_NCHUNK = _B_PER_W // _CHUNK


def _sc_gather_body(idx_hbm, par_hbm, code_hbm, hier_hbm, ec_out, ep_out,
                    idx_v, par_v, c0, c1, c2, c3, h0, h1, h2, h3,
                    sc0, sc1, sh0):
    wid = lax.axis_index("s") * _NC + lax.axis_index("c")
    base = wid * _B_PER_W
    stage = []
    for j in range(_NCHUNK):
        src = pl.ds(base + j * _CHUNK, _CHUNK)
        stage.append(pltpu.async_copy(idx_hbm.at[src], idx_v.at[j], sc0))
        stage.append(pltpu.async_copy(par_hbm.at[src], par_v.at[j], sc1))
    for cp in stage:
        cp.wait()
    for ref in (idx_v, par_v):
        for j in range(_NCHUNK):
            for k in range(_CHUNK // 16):
                s = pl.ds(k * 16, 16)
                ref[j, s] = ref[j, s] * 4
    cbufs = (c0, c1, c2, c3)
    hbufs = (h0, h1, h2, h3)
    cps_c = [pltpu.async_copy(code_hbm.at[idx_v.at[j]], cbufs[j], sc0)
             for j in range(_NCHUNK)]
    cps_h = [pltpu.async_copy(hier_hbm.at[par_v.at[j]], hbufs[j], sh0)
             for j in range(_NCHUNK)]
    for j in range(_NCHUNK):
        rows = pl.ds(base + j * _CHUNK, _CHUNK)
        lanes = pl.ds(0, EMBED_DIM)
        cps_c[j].wait()
        pltpu.sync_copy(cbufs[j], ec_out.at[rows, lanes])
        cps_h[j].wait()
        pltpu.sync_copy(hbufs[j], ep_out.at[rows, lanes])


_sc_gather = functools.partial(
    pl.kernel,
    out_type=[
        jax.ShapeDtypeStruct((BATCH, PAD_DIM), jnp.float32),
        jax.ShapeDtypeStruct((BATCH, PAD_DIM), jnp.float32),
    ],
    mesh=plsc.VectorSubcoreMesh(core_axis_name="c", subcore_axis_name="s"),
    compiler_params=pltpu.CompilerParams(use_tc_tiling_on_sc=False),
    scratch_types=[
        pltpu.VMEM((_NCHUNK, _CHUNK), jnp.int32),
        pltpu.VMEM((_NCHUNK, _CHUNK), jnp.int32),
        pltpu.VMEM((_CHUNK, EMBED_DIM), jnp.float32),
        pltpu.VMEM((_CHUNK, EMBED_DIM), jnp.float32),
        pltpu.VMEM((_CHUNK, EMBED_DIM), jnp.float32),
        pltpu.VMEM((_CHUNK, EMBED_DIM), jnp.float32),
        pltpu.VMEM((_CHUNK, EMBED_DIM), jnp.float32),
        pltpu.VMEM((_CHUNK, EMBED_DIM), jnp.float32),
        pltpu.VMEM((_CHUNK, EMBED_DIM), jnp.float32),
        pltpu.VMEM((_CHUNK, EMBED_DIM), jnp.float32),
        pltpu.SemaphoreType.DMA,
        pltpu.SemaphoreType.DMA,
        pltpu.SemaphoreType.DMA,
    ],
)(_sc_gather_body)


_TP_BLK = 16384


def _tp_body(tt_ref, o_ref):
    t = jax.lax.dot_general(
        tt_ref[...], jnp.eye(EMBED_DIM, dtype=jnp.float32),
        (((0,), (0,)), ((), ())), preferred_element_type=jnp.float32)
    o_ref[...] = jnp.concatenate(
        [t, jnp.zeros((_TP_BLK, PAD_DIM - EMBED_DIM), jnp.float32)], axis=1)


def _tc_transpose_pad(table_t, vocab):
    grid = ((vocab + _TP_BLK - 1) // _TP_BLK,)
    return pl.pallas_call(
        _tp_body,
        grid=grid,
        in_specs=[pl.BlockSpec((EMBED_DIM, _TP_BLK), lambda i: (0, i))],
        out_specs=pl.BlockSpec((_TP_BLK, PAD_DIM), lambda i: (i, 0)),
        out_shape=jax.ShapeDtypeStruct((vocab, PAD_DIM), jnp.float32),
    )(table_t)


_MM_BLK = 8192


def _mm_body(ec_ref, ep_ref, w1_ref, w2_ref, b_ref, o_ref):
    ec = ec_ref[:, : EMBED_DIM]
    ep = ep_ref[:, : EMBED_DIM]
    acc = jax.lax.dot_general(
        w1_ref[...], ec, (((0,), (1,)), ((), ())),
        preferred_element_type=jnp.float32)
    acc += jax.lax.dot_general(
        w2_ref[...], ep, (((0,), (1,)), ((), ())),
        preferred_element_type=jnp.float32)
    o_ref[...] = acc + b_ref[...]


def _tc_project(ec, ep, w1, w2, bcol):
    grid = (BATCH // _MM_BLK,)
    return pl.pallas_call(
        _mm_body,
        grid=grid,
        in_specs=[
            pl.BlockSpec((_MM_BLK, PAD_DIM), lambda i: (i, 0)),
            pl.BlockSpec((_MM_BLK, PAD_DIM), lambda i: (i, 0)),
            pl.BlockSpec((EMBED_DIM, OUT_DIM), lambda i: (0, 0)),
            pl.BlockSpec((EMBED_DIM, OUT_DIM), lambda i: (0, 0)),
            pl.BlockSpec((OUT_DIM, 1), lambda i: (0, 0)),
        ],
        out_specs=pl.BlockSpec((OUT_DIM, _MM_BLK), lambda i: (0, i)),
        out_shape=jax.ShapeDtypeStruct((OUT_DIM, BATCH), jnp.float32),
    )(ec, ep, w1, w2, bcol)


@jax.jit
def kernel(indices, parents, code_table, hier_table, W, b):
    code_lin = _tc_transpose_pad(code_table.T, VOCAB).reshape(
        4 * VOCAB, EMBED_DIM)
    hier_lin = _tc_transpose_pad(hier_table.T, HIER_VOCAB).reshape(
        4 * HIER_VOCAB, EMBED_DIM)
    ec, ep = _sc_gather(indices, parents, code_lin, hier_lin)
    w1 = W[:EMBED_DIM]
    w2 = W[EMBED_DIM:]
    out_t = _tc_project(ec, ep, w1, w2, b.reshape(OUT_DIM, 1))
    return out_t.T


# speedup vs baseline: 1.4850x; 1.0192x over previous
import functools

import jax
import jax.numpy as jnp
from jax import lax
from jax.experimental import pallas as pl
from jax.experimental.pallas import tpu as pltpu
from jax.experimental.pallas import tpu_sc as plsc

BATCH = 16384
VOCAB = 100000
HIER_VOCAB = 10000
EMBED_DIM = 32
OUT_DIM = 32
PAD_DIM = 128

_NC = 2
_NS = 16
_NW = _NC * _NS
_B_PER_W = BATCH // _NW
_CHUNK = 128
_NCHUNK = _B_PER_W // _CHUNK


def _sc_gather_body(idx_hbm, par_hbm, code_hbm, hier_hbm, ec_out, ep_out,
                    idx_v, par_v, c0, c1, c2, c3, h0, h1, h2, h3,
                    sc0, sc1, sh0):
    wid = lax.axis_index("s") * _NC + lax.axis_index("c")
    base = wid * _B_PER_W
    stage = []
    for j in range(_NCHUNK):
        src = pl.ds(base + j * _CHUNK, _CHUNK)
        stage.append(pltpu.async_copy(idx_hbm.at[src], idx_v.at[j], sc0))
        stage.append(pltpu.async_copy(par_hbm.at[src], par_v.at[j], sc1))
    for cp in stage:
        cp.wait()
    for ref in (idx_v, par_v):
        for j in range(_NCHUNK):
            for k in range(_CHUNK // 16):
                s = pl.ds(k * 16, 16)
                ref[j, s] = ref[j, s] * 4
    cbufs = (c0, c1, c2, c3)
    hbufs = (h0, h1, h2, h3)
    cps_c = [pltpu.async_copy(code_hbm.at[idx_v.at[j]], cbufs[j], sc0)
             for j in range(_NCHUNK)]
    cps_h = [pltpu.async_copy(hier_hbm.at[par_v.at[j]], hbufs[j], sh0)
             for j in range(_NCHUNK)]
    for j in range(_NCHUNK):
        rows = pl.ds(base + j * _CHUNK, _CHUNK)
        lanes = pl.ds(0, EMBED_DIM)
        cps_c[j].wait()
        pltpu.sync_copy(cbufs[j], ec_out.at[rows, lanes])
        cps_h[j].wait()
        pltpu.sync_copy(hbufs[j], ep_out.at[rows, lanes])


_sc_gather = functools.partial(
    pl.kernel,
    out_type=[
        jax.ShapeDtypeStruct((BATCH, PAD_DIM), jnp.float32),
        jax.ShapeDtypeStruct((BATCH, PAD_DIM), jnp.float32),
    ],
    mesh=plsc.VectorSubcoreMesh(core_axis_name="c", subcore_axis_name="s"),
    compiler_params=pltpu.CompilerParams(use_tc_tiling_on_sc=False),
    scratch_types=[
        pltpu.VMEM((_NCHUNK, _CHUNK), jnp.int32),
        pltpu.VMEM((_NCHUNK, _CHUNK), jnp.int32),
        pltpu.VMEM((_CHUNK, EMBED_DIM), jnp.float32),
        pltpu.VMEM((_CHUNK, EMBED_DIM), jnp.float32),
        pltpu.VMEM((_CHUNK, EMBED_DIM), jnp.float32),
        pltpu.VMEM((_CHUNK, EMBED_DIM), jnp.float32),
        pltpu.VMEM((_CHUNK, EMBED_DIM), jnp.float32),
        pltpu.VMEM((_CHUNK, EMBED_DIM), jnp.float32),
        pltpu.VMEM((_CHUNK, EMBED_DIM), jnp.float32),
        pltpu.VMEM((_CHUNK, EMBED_DIM), jnp.float32),
        pltpu.SemaphoreType.DMA,
        pltpu.SemaphoreType.DMA,
        pltpu.SemaphoreType.DMA,
    ],
)(_sc_gather_body)


_TP_BLK = 16384


_N_CODE_STEPS = (VOCAB + _TP_BLK - 1) // _TP_BLK


def _tp_block(tt_ref):
    t = jax.lax.dot_general(
        tt_ref[...], jnp.eye(EMBED_DIM, dtype=jnp.float32),
        (((0,), (0,)), ((), ())), preferred_element_type=jnp.float32)
    return jnp.concatenate(
        [t, jnp.zeros((_TP_BLK, PAD_DIM - EMBED_DIM), jnp.float32)], axis=1)


def _tp_body(code_ref, hier_ref, oc_ref, oh_ref):
    i = pl.program_id(0)

    @pl.when(i < _N_CODE_STEPS)
    def _():
        oc_ref[...] = _tp_block(code_ref)

    @pl.when(i == _N_CODE_STEPS)
    def _():
        oh_ref[...] = _tp_block(hier_ref)


def _tc_transpose_pad(code_t, hier_t):
    last = _N_CODE_STEPS - 1
    return pl.pallas_call(
        _tp_body,
        grid=(_N_CODE_STEPS + 1,),
        in_specs=[
            pl.BlockSpec((EMBED_DIM, _TP_BLK), lambda i: (0, jnp.minimum(i, last))),
            pl.BlockSpec((EMBED_DIM, _TP_BLK), lambda i: (0, 0)),
        ],
        out_specs=[
            pl.BlockSpec((_TP_BLK, PAD_DIM), lambda i: (jnp.minimum(i, last), 0)),
            pl.BlockSpec((_TP_BLK, PAD_DIM), lambda i: (0, 0)),
        ],
        out_shape=[
            jax.ShapeDtypeStruct((VOCAB, PAD_DIM), jnp.float32),
            jax.ShapeDtypeStruct((HIER_VOCAB, PAD_DIM), jnp.float32),
        ],
    )(code_t, hier_t)


_MM_BLK = 8192


def _mm_body(ec_ref, ep_ref, w1_ref, w2_ref, b_ref, o_ref):
    ec = ec_ref[:, : EMBED_DIM]
    ep = ep_ref[:, : EMBED_DIM]
    acc = jax.lax.dot_general(
        w1_ref[...], ec, (((0,), (1,)), ((), ())),
        preferred_element_type=jnp.float32)
    acc += jax.lax.dot_general(
        w2_ref[...], ep, (((0,), (1,)), ((), ())),
        preferred_element_type=jnp.float32)
    o_ref[...] = acc + b_ref[...]


def _tc_project(ec, ep, w1, w2, bcol):
    grid = (BATCH // _MM_BLK,)
    return pl.pallas_call(
        _mm_body,
        grid=grid,
        in_specs=[
            pl.BlockSpec((_MM_BLK, PAD_DIM), lambda i: (i, 0)),
            pl.BlockSpec((_MM_BLK, PAD_DIM), lambda i: (i, 0)),
            pl.BlockSpec((EMBED_DIM, OUT_DIM), lambda i: (0, 0)),
            pl.BlockSpec((EMBED_DIM, OUT_DIM), lambda i: (0, 0)),
            pl.BlockSpec((OUT_DIM, 1), lambda i: (0, 0)),
        ],
        out_specs=pl.BlockSpec((OUT_DIM, _MM_BLK), lambda i: (0, i)),
        out_shape=jax.ShapeDtypeStruct((OUT_DIM, BATCH), jnp.float32),
    )(ec, ep, w1, w2, bcol)


@jax.jit
def kernel(indices, parents, code_table, hier_table, W, b):
    code_pad, hier_pad = _tc_transpose_pad(code_table.T, hier_table.T)
    code_lin = code_pad.reshape(4 * VOCAB, EMBED_DIM)
    hier_lin = hier_pad.reshape(4 * HIER_VOCAB, EMBED_DIM)
    ec, ep = _sc_gather(indices, parents, code_lin, hier_lin)
    w1 = W[:EMBED_DIM]
    w2 = W[EMBED_DIM:]
    out_t = _tc_project(ec, ep, w1, w2, b.reshape(OUT_DIM, 1))
    return out_t.T
